# Initial kernel scaffold; baseline (speedup 1.0000x reference)
import functools

import jax
import jax.numpy as jnp
from jax import lax
from jax.experimental import pallas as pl
from jax.experimental.pallas import tpu as pltpu
from jax.experimental.pallas import tpu_sc as plsc

N = 10000
E = 320000
H = 2
C = 64
HC = 128
G = 64

NC = 2
NS = 16
NW = NC * NS
EW = E // NW
B = 200
NBLK = EW // B
RPT = N // NS
QW = 144
KVW = 256


def _prep_call(x, wq, bq, wkv, bkv, ws, bs, blk):
    m = x.shape[0]

    def body(x_ref, wq_ref, bq_ref, wkv_ref, bkv_ref, ws_ref, bs_ref,
             q_ref, kv_ref, s_ref):
        xb = x_ref[...]
        q_ref[...] = jnp.dot(xb, wq_ref[...],
                             preferred_element_type=jnp.float32) + bq_ref[...]
        kv_ref[...] = jnp.dot(xb, wkv_ref[...],
                              preferred_element_type=jnp.float32) + bkv_ref[...]
        s_ref[...] = jnp.dot(xb, ws_ref[...],
                             preferred_element_type=jnp.float32) + bs_ref[...]

    return pl.pallas_call(
        body,
        grid=(m // blk,),
        in_specs=[
            pl.BlockSpec((blk, HC), lambda i: (i, 0)),
            pl.BlockSpec((HC, QW), lambda i: (0, 0)),
            pl.BlockSpec((1, QW), lambda i: (0, 0)),
            pl.BlockSpec((HC, KVW), lambda i: (0, 0)),
            pl.BlockSpec((1, KVW), lambda i: (0, 0)),
            pl.BlockSpec((HC, HC), lambda i: (0, 0)),
            pl.BlockSpec((1, HC), lambda i: (0, 0)),
        ],
        out_specs=[
            pl.BlockSpec((blk, QW), lambda i: (i, 0)),
            pl.BlockSpec((blk, KVW), lambda i: (i, 0)),
            pl.BlockSpec((blk, HC), lambda i: (i, 0)),
        ],
        out_shape=[
            jax.ShapeDtypeStruct((m, QW), jnp.float32),
            jax.ShapeDtypeStruct((m, KVW), jnp.float32),
            jax.ShapeDtypeStruct((m, HC), jnp.float32),
        ],
    )(x, wq, bq, wkv, bkv, ws, bs)


def _edge_call(qp, kv, ea16, src, dst):
    mesh = plsc.VectorSubcoreMesh(core_axis_name="c", subcore_axis_name="s")

    @functools.partial(
        pl.kernel,
        out_type=(
            jax.ShapeDtypeStruct((NC * N, HC), jnp.float32),
            jax.ShapeDtypeStruct((NC * N, 16), jnp.float32),
        ),
        mesh=mesh,
        scratch_types=[
            pltpu.VMEM((B,), jnp.int32),
            pltpu.VMEM((B,), jnp.int32),
            pltpu.VMEM((B, QW), jnp.float32),
            pltpu.VMEM((B, KVW), jnp.float32),
            pltpu.VMEM((B, 16), jnp.float32),
            pltpu.VMEM((B, HC), jnp.float32),
            pltpu.VMEM((B, 16), jnp.float32),
            pltpu.VMEM_SHARED((N, HC), jnp.float32),
            pltpu.VMEM_SHARED((N, 16), jnp.float32),
            pltpu.SemaphoreType.DMA,
        ],
    )
    def k(qp_hbm, kv_hbm, ea_hbm, src_hbm, dst_hbm, acc_hbm, den_hbm,
          sidx, didx, qrows, kvrows, earows, outrows, denrows,
          acc_sh, den_sh, sem):
        core = lax.axis_index("c")
        sub = lax.axis_index("s")
        wid = sub * NC + core
        io16 = lax.iota(jnp.int32, 16)
        z16 = jnp.zeros((16,), jnp.float32)

        def zrow(j, carry):
            for c in range(HC // 16):
                outrows[j, pl.ds(c * 16, 16)] = z16
            denrows[j, :] = z16
            return carry

        lax.fori_loop(0, B, zrow, 0)
        rbase = sub * RPT
        for p in range(RPT // B):
            pltpu.sync_copy(outrows, acc_sh.at[pl.ds(rbase + p * B, B)])
            pltpu.sync_copy(denrows, den_sh.at[pl.ds(rbase + p * B, B)])
        tail = RPT % B
        if tail:
            toff = rbase + (RPT // B) * B
            pltpu.sync_copy(outrows.at[pl.ds(0, tail)],
                            acc_sh.at[pl.ds(toff, tail)])
            pltpu.sync_copy(denrows.at[pl.ds(0, tail)],
                            den_sh.at[pl.ds(toff, tail)])
        plsc.subcore_barrier()

        def blk_body(blk, carry):
            base = wid * EW + blk * B
            pltpu.sync_copy(src_hbm.at[pl.ds(base, B)], sidx)
            pltpu.sync_copy(dst_hbm.at[pl.ds(base, B)], didx)
            pltpu.async_copy(qp_hbm.at[didx], qrows, sem).wait()
            pltpu.async_copy(kv_hbm.at[sidx], kvrows, sem).wait()
            pltpu.sync_copy(ea_hbm.at[pl.ds(base, B)], earows)

            def edge(i, ecarry):
                a0 = qrows[i, pl.ds(0, 16)] * kvrows[i, pl.ds(0, 16)]
                a1 = qrows[i, pl.ds(64, 16)] * kvrows[i, pl.ds(64, 16)]
                for c in range(1, 4):
                    a0 = a0 + (qrows[i, pl.ds(c * 16, 16)]
                               * kvrows[i, pl.ds(c * 16, 16)])
                    a1 = a1 + (qrows[i, pl.ds(64 + c * 16, 16)]
                               * kvrows[i, pl.ds(64 + c * 16, 16)])
                evec = earows[i, :]
                pe = qrows[i, pl.ds(128, 16)] * evec
                a0 = a0 + jnp.where(io16 < 4, pe, 0.0)
                a1 = a1 + jnp.where((io16 >= 4) & (io16 < 8), pe, 0.0)
                s0v = jnp.exp(jnp.full((16,), jnp.sum(a0) * 0.125,
                                       jnp.float32))
                s1v = jnp.exp(jnp.full((16,), jnp.sum(a1) * 0.125,
                                       jnp.float32))
                for c in range(4):
                    outrows[i, pl.ds(c * 16, 16)] = (
                        s0v * kvrows[i, pl.ds(128 + c * 16, 16)])
                    outrows[i, pl.ds(64 + c * 16, 16)] = (
                        s1v * kvrows[i, pl.ds(192 + c * 16, 16)])
                shead = jnp.where(io16 < 4, s0v,
                                  jnp.where(io16 < 8, s1v, 0.0))
                srow = jnp.where(io16 == 8, s0v,
                                 jnp.where(io16 == 9, s1v, 0.0))
                denrows[i, :] = evec * shead + srow
                return ecarry

            lax.fori_loop(0, B, edge, 0)
            pltpu.sync_copy(outrows, acc_sh.at[didx], add=True)
            pltpu.sync_copy(denrows, den_sh.at[didx], add=True)
            return carry

        lax.fori_loop(0, NBLK, blk_body, 0)
        plsc.subcore_barrier()

        obase = core * N + rbase
        for p in range(RPT // B):
            pltpu.sync_copy(acc_sh.at[pl.ds(rbase + p * B, B)], outrows)
            pltpu.sync_copy(outrows, acc_hbm.at[pl.ds(obase + p * B, B)])
            pltpu.sync_copy(den_sh.at[pl.ds(rbase + p * B, B)], denrows)
            pltpu.sync_copy(denrows, den_hbm.at[pl.ds(obase + p * B, B)])
        if tail:
            toff = (RPT // B) * B
            pltpu.sync_copy(acc_sh.at[pl.ds(rbase + toff, tail)],
                            outrows.at[pl.ds(0, tail)])
            pltpu.sync_copy(outrows.at[pl.ds(0, tail)],
                            acc_hbm.at[pl.ds(obase + toff, tail)])
            pltpu.sync_copy(den_sh.at[pl.ds(rbase + toff, tail)],
                            denrows.at[pl.ds(0, tail)])
            pltpu.sync_copy(denrows.at[pl.ds(0, tail)],
                            den_hbm.at[pl.ds(obase + toff, tail)])

    return k(qp, kv, ea16, src, dst)


def _combine_call(acc0, acc1, den0, den1, skip, wt, relu, blk):
    def body(a0_ref, a1_ref, d0_ref, d1_ref, s_ref, wt_ref, o_ref):
        num = a0_ref[...] + a1_ref[...]
        t16 = d0_ref[...] + d1_ref[...]
        num = num + jnp.dot(t16[:, 0:8], wt_ref[...],
                            preferred_element_type=jnp.float32)
        den = jnp.where(
            lax.broadcasted_iota(jnp.int32, (blk, HC), 1) < C,
            t16[:, 8:9], t16[:, 9:10])
        h = jnp.where(den > 0.0, num / den, 0.0) + s_ref[...]
        if relu:
            h = jnp.maximum(h, 0.0)
        o_ref[...] = h

    return pl.pallas_call(
        body,
        grid=(N // blk,),
        in_specs=[
            pl.BlockSpec((blk, HC), lambda i: (i, 0)),
            pl.BlockSpec((blk, HC), lambda i: (i, 0)),
            pl.BlockSpec((blk, 16), lambda i: (i, 0)),
            pl.BlockSpec((blk, 16), lambda i: (i, 0)),
            pl.BlockSpec((blk, HC), lambda i: (i, 0)),
            pl.BlockSpec((8, HC), lambda i: (0, 0)),
        ],
        out_specs=pl.BlockSpec((blk, HC), lambda i: (i, 0)),
        out_shape=jax.ShapeDtypeStruct((N, HC), jnp.float32),
    )(acc0, acc1, den0, den1, skip, wt)


def _pool_call(h, batch3d, blk):
    def body(b_ref, h_ref, sum_ref, cnt_ref):
        @pl.when(pl.program_id(0) == 0)
        def _init():
            sum_ref[...] = jnp.zeros((G, HC), jnp.float32)
            cnt_ref[...] = jnp.zeros((G, HC), jnp.float32)

        oh = (b_ref[0, 0, :][None, :]
              == lax.broadcasted_iota(jnp.int32, (G, blk), 0)
              ).astype(jnp.float32)
        sum_ref[...] += jnp.dot(oh, h_ref[...],
                                preferred_element_type=jnp.float32)
        cnt_ref[...] += jnp.broadcast_to(
            jnp.sum(oh, axis=1, keepdims=True), (G, HC))

    return pl.pallas_call(
        body,
        grid=(N // blk,),
        in_specs=[
            pl.BlockSpec((1, 1, blk), lambda i: (i, 0, 0)),
            pl.BlockSpec((blk, HC), lambda i: (i, 0)),
        ],
        out_specs=[
            pl.BlockSpec((G, HC), lambda i: (0, 0)),
            pl.BlockSpec((G, HC), lambda i: (0, 0)),
        ],
        out_shape=[
            jax.ShapeDtypeStruct((G, HC), jnp.float32),
            jax.ShapeDtypeStruct((G, HC), jnp.float32),
        ],
    )(batch3d, h)


def _mlp_call(sums, cnt, w1, b1, w2, b2, w3, b3):
    def body(s_ref, c_ref, w1_ref, b1_ref, w2_ref, b2_ref, w3_ref, b3_ref,
             o_ref):
        gm = s_ref[...] / jnp.maximum(c_ref[...], 1.0)
        g = jnp.maximum(jnp.dot(gm, w1_ref[...],
                                preferred_element_type=jnp.float32)
                        + b1_ref[...], 0.0)
        g = jnp.maximum(jnp.dot(g, w2_ref[...],
                                preferred_element_type=jnp.float32)
                        + b2_ref[...], 0.0)
        o_ref[...] = jnp.dot(g, w3_ref[...],
                             preferred_element_type=jnp.float32) + b3_ref[...]

    return pl.pallas_call(
        body,
        out_shape=jax.ShapeDtypeStruct((G, 16), jnp.float32),
    )(sums, cnt, w1, b1.reshape(1, -1), w2, b2.reshape(1, -1),
      w3, b3.reshape(1, -1))


def _make_m(we):
    t = jnp.transpose(we.reshape(4, H, C), (1, 2, 0))
    m = t[:, :, None, :] * jnp.eye(H, dtype=jnp.float32)[:, None, :, None]
    return m.reshape(HC, H * 4)


def _layer(x_in, src, dst, ea16, wq, bq, wk, bk, wv, bv, we, ws, bs, relu):
    m = _make_m(we)
    wq_ext = jnp.concatenate(
        [wq, wq @ m, jnp.zeros((HC, 8), jnp.float32)], axis=1)
    bq_ext = jnp.concatenate(
        [bq, bq @ m, jnp.zeros((8,), jnp.float32)]).reshape(1, QW)
    wkv = jnp.concatenate([wk, wv], axis=1)
    bkv = jnp.concatenate([bk, bv]).reshape(1, KVW)
    qp, kv, skip = _prep_call(x_in, wq_ext, bq_ext, wkv, bkv, ws,
                              bs.reshape(1, HC), 400)
    acc, den = _edge_call(qp, kv, ea16, src, dst)
    return _combine_call(acc[:N], acc[N:], den[:N], den[N:], skip,
                         jnp.transpose(m), relu, 1000)


def kernel(x, edge_index, batch, edge_attr,
           l1_Wq, l1_bq, l1_Wk, l1_bk, l1_Wv, l1_bv, l1_We, l1_Ws, l1_bs,
           l2_Wq, l2_bq, l2_Wk, l2_bk, l2_Wv, l2_bv, l2_We, l2_Ws, l2_bs,
           ge_W1, ge_b1, ge_W2, ge_b2, ge_W3, ge_b3):
    src = edge_index[0]
    dst = edge_index[1]
    ea16 = jnp.concatenate(
        [edge_attr, edge_attr, jnp.zeros((E, 8), jnp.float32)], axis=1)
    h = _layer(x, src, dst, ea16, l1_Wq, l1_bq, l1_Wk, l1_bk, l1_Wv, l1_bv,
               l1_We, l1_Ws, l1_bs, True)
    h = _layer(h, src, dst, ea16, l2_Wq, l2_bq, l2_Wk, l2_bk, l2_Wv, l2_bv,
               l2_We, l2_Ws, l2_bs, False)
    sums, cnt = _pool_call(h, batch.reshape(N // 1000, 1, 1000), 1000)
    g = _mlp_call(sums, cnt, ge_W1, ge_b1, ge_W2, ge_b2, ge_W3, ge_b3)
    return (h, g)


# speedup vs baseline: 14.8687x; 14.8687x over previous
import functools

import jax
import jax.numpy as jnp
from jax import lax
from jax.experimental import pallas as pl
from jax.experimental.pallas import tpu as pltpu
from jax.experimental.pallas import tpu_sc as plsc

N = 10000
E = 320000
H = 2
C = 64
HC = 128
G = 64

NC = 2
NS = 16
EW = E // NS
B = 160
KC = 2
MC = 80
NBLK = EW // B
NP = 10240
RPT = NP // NS
DR = 160
QW = 80
AW = 80


def _lperm(v, perm):
    return lax.gather(
        v, perm[:, None],
        lax.GatherDimensionNumbers(offset_dims=(), collapsed_slice_dims=(0,),
                                   start_index_map=(0,)),
        (1,), mode=lax.GatherScatterMode.PROMISE_IN_BOUNDS)


def _prep_call(x, wq, bq, wkv, bkv, ws, bs, blk):
    def body(x_ref, wq_ref, bq_ref, wkv_ref, bkv_ref, ws_ref, bs_ref,
             q_ref, kv_ref, s_ref):
        xb = x_ref[...]
        q_ref[...] = jnp.dot(xb, wq_ref[0],
                             preferred_element_type=jnp.float32) + bq_ref[0]
        kv_ref[...] = jnp.dot(xb, wkv_ref[0],
                              preferred_element_type=jnp.float32) + bkv_ref[0]
        s_ref[...] = jnp.dot(xb, ws_ref[...],
                             preferred_element_type=jnp.float32) + bs_ref[...]

    nblk = N // blk
    return pl.pallas_call(
        body,
        grid=(H, nblk),
        in_specs=[
            pl.BlockSpec((blk, HC), lambda h, i: (i, 0)),
            pl.BlockSpec((1, HC, QW), lambda h, i: (h, 0, 0)),
            pl.BlockSpec((1, 1, QW), lambda h, i: (h, 0, 0)),
            pl.BlockSpec((1, HC, HC), lambda h, i: (h, 0, 0)),
            pl.BlockSpec((1, 1, HC), lambda h, i: (h, 0, 0)),
            pl.BlockSpec((HC, HC), lambda h, i: (0, 0)),
            pl.BlockSpec((1, HC), lambda h, i: (0, 0)),
        ],
        out_specs=[
            pl.BlockSpec((blk, QW), lambda h, i: (h * nblk + i, 0)),
            pl.BlockSpec((blk, HC), lambda h, i: (h * nblk + i, 0)),
            pl.BlockSpec((blk, HC), lambda h, i: (i, 0)),
        ],
        out_shape=[
            jax.ShapeDtypeStruct((H * N, QW), jnp.float32),
            jax.ShapeDtypeStruct((H * N, HC), jnp.float32),
            jax.ShapeDtypeStruct((N, HC), jnp.float32),
        ],
    )(x, wq, bq, wkv, bkv, ws, bs)


def _edge_call(qp, kv, ea16, src, dst):
    mesh = plsc.VectorSubcoreMesh(core_axis_name="c", subcore_axis_name="s")

    @functools.partial(
        pl.kernel,
        out_type=jax.ShapeDtypeStruct((NC * NP, AW), jnp.float32),
        mesh=mesh,
        scratch_types=[
            pltpu.VMEM((KC, MC), jnp.int32),
            pltpu.VMEM((KC, MC), jnp.int32),
            pltpu.VMEM((KC, MC), jnp.int32),
            pltpu.VMEM((B, QW), jnp.float32),
            pltpu.VMEM((B, HC), jnp.float32),
            pltpu.VMEM((B, 16), jnp.float32),
            pltpu.VMEM((B, AW), jnp.float32),
            pltpu.VMEM_SHARED((NP, AW), jnp.float32),
            pltpu.SemaphoreType.DMA,
        ],
        compiler_params=pltpu.CompilerParams(use_tc_tiling_on_sc=False),
    )
    def k(qp_hbm, kv_hbm, ea_hbm, src_hbm, dst_hbm, acc_hbm,
          sidx, didx, didx_raw, qrows, kvrows, earows, outrows,
          acc_sh, sem):
        core = lax.axis_index("c")
        sub = lax.axis_index("s")
        io16 = lax.iota(jnp.int32, 16)
        z16 = jnp.zeros((16,), jnp.float32)
        hoff = core * N

        def zrow(j, carry):
            for c in range(AW // 16):
                outrows[j, pl.ds(c * 16, 16)] = z16
            return carry

        lax.fori_loop(0, B, zrow, 0)
        rbase = sub * RPT
        for p in range(RPT // DR):
            pltpu.sync_copy(outrows.at[pl.ds(0, DR)],
                            acc_sh.at[pl.ds(rbase + p * DR, DR)])
        plsc.subcore_barrier()

        def blk_body(blk, carry):
            base = sub * EW + blk * B
            rowoff = base // MC
            pltpu.sync_copy(src_hbm.at[pl.ds(rowoff, KC)], sidx)
            pltpu.sync_copy(dst_hbm.at[pl.ds(rowoff, KC)], didx_raw)
            for r in range(KC):
                for c in range(MC // 16):
                    sidx[r, pl.ds(c * 16, 16)] = (
                        sidx[r, pl.ds(c * 16, 16)] + hoff)
                    didx[r, pl.ds(c * 16, 16)] = (
                        didx_raw[r, pl.ds(c * 16, 16)] + hoff)
            for r in range(KC):
                pltpu.async_copy(qp_hbm.at[didx.at[r]],
                                 qrows.at[pl.ds(r * MC, MC)], sem).wait()
                pltpu.async_copy(kv_hbm.at[sidx.at[r]],
                                 kvrows.at[pl.ds(r * MC, MC)], sem).wait()
            pltpu.sync_copy(ea_hbm.at[pl.ds(base, B)], earows)

            def edge(i, ecarry):
                a = qrows[i, pl.ds(0, 16)] * kvrows[i, pl.ds(0, 16)]
                for c in range(1, 4):
                    a = a + (qrows[i, pl.ds(c * 16, 16)]
                             * kvrows[i, pl.ds(c * 16, 16)])
                evec = earows[i, :]
                pe = qrows[i, pl.ds(C, 16)] * evec
                a = a + jnp.where(io16 < 4, pe, 0.0)
                for sh in (8, 4, 2, 1):
                    a = a + _lperm(a, io16 ^ sh)
                sv = jnp.exp(a * 0.125)
                for c in range(4):
                    outrows[i, pl.ds(c * 16, 16)] = (
                        sv * kvrows[i, pl.ds(C + c * 16, 16)])
                shead = jnp.where(io16 < 4, sv, 0.0)
                outrows[i, pl.ds(C, 16)] = (evec * shead
                                            + jnp.where(io16 == 8, sv, 0.0))
                return ecarry

            lax.fori_loop(0, B, edge, 0)
            for r in range(KC):
                pltpu.sync_copy(outrows.at[pl.ds(r * MC, MC)],
                                acc_sh.at[didx_raw.at[r]], add=True)
            return carry

        lax.fori_loop(0, NBLK, blk_body, 0)
        plsc.subcore_barrier()

        obase = core * NP + rbase
        for p in range(RPT // DR):
            pltpu.sync_copy(acc_sh.at[pl.ds(rbase + p * DR, DR)],
                            outrows.at[pl.ds(0, DR)])
            pltpu.sync_copy(outrows.at[pl.ds(0, DR)],
                            acc_hbm.at[pl.ds(obase + p * DR, DR)])

    return k(qp, kv, ea16, src.reshape(E // MC, MC),
             dst.reshape(E // MC, MC))


def _combine_call(acc0, acc1, skip, wt0, wt1, relu, blk):
    def body(a0_ref, a1_ref, s_ref, wt0_ref, wt1_ref, o_ref):
        a0 = a0_ref[...]
        a1 = a1_ref[...]
        n0 = a0[:, 0:C] + jnp.dot(a0[:, C:C + 4], wt0_ref[...],
                                  preferred_element_type=jnp.float32)
        n1 = a1[:, 0:C] + jnp.dot(a1[:, C:C + 4], wt1_ref[...],
                                  preferred_element_type=jnp.float32)
        s0 = a0[:, C + 8:C + 9]
        s1 = a1[:, C + 8:C + 9]
        h0 = jnp.where(s0 > 0.0, n0 / s0, 0.0)
        h1 = jnp.where(s1 > 0.0, n1 / s1, 0.0)
        h = jnp.concatenate([h0, h1], axis=1) + s_ref[...]
        if relu:
            h = jnp.maximum(h, 0.0)
        o_ref[...] = h

    return pl.pallas_call(
        body,
        grid=(N // blk,),
        in_specs=[
            pl.BlockSpec((blk, AW), lambda i: (i, 0)),
            pl.BlockSpec((blk, AW), lambda i: (i, 0)),
            pl.BlockSpec((blk, HC), lambda i: (i, 0)),
            pl.BlockSpec((4, C), lambda i: (0, 0)),
            pl.BlockSpec((4, C), lambda i: (0, 0)),
        ],
        out_specs=pl.BlockSpec((blk, HC), lambda i: (i, 0)),
        out_shape=jax.ShapeDtypeStruct((N, HC), jnp.float32),
    )(acc0, acc1, skip, wt0, wt1)


def _pool_call(h, batch3d, blk):
    def body(b_ref, h_ref, sum_ref, cnt_ref):
        @pl.when(pl.program_id(0) == 0)
        def _init():
            sum_ref[...] = jnp.zeros((G, HC), jnp.float32)
            cnt_ref[...] = jnp.zeros((G, HC), jnp.float32)

        oh = (b_ref[0, 0, :][None, :]
              == lax.broadcasted_iota(jnp.int32, (G, blk), 0)
              ).astype(jnp.float32)
        sum_ref[...] += jnp.dot(oh, h_ref[...],
                                preferred_element_type=jnp.float32)
        cnt_ref[...] += jnp.broadcast_to(
            jnp.sum(oh, axis=1, keepdims=True), (G, HC))

    return pl.pallas_call(
        body,
        grid=(N // blk,),
        in_specs=[
            pl.BlockSpec((1, 1, blk), lambda i: (i, 0, 0)),
            pl.BlockSpec((blk, HC), lambda i: (i, 0)),
        ],
        out_specs=[
            pl.BlockSpec((G, HC), lambda i: (0, 0)),
            pl.BlockSpec((G, HC), lambda i: (0, 0)),
        ],
        out_shape=[
            jax.ShapeDtypeStruct((G, HC), jnp.float32),
            jax.ShapeDtypeStruct((G, HC), jnp.float32),
        ],
    )(batch3d, h)


def _mlp_call(sums, cnt, w1, b1, w2, b2, w3, b3):
    def body(s_ref, c_ref, w1_ref, b1_ref, w2_ref, b2_ref, w3_ref, b3_ref,
             o_ref):
        gm = s_ref[...] / jnp.maximum(c_ref[...], 1.0)
        g = jnp.maximum(jnp.dot(gm, w1_ref[...],
                                preferred_element_type=jnp.float32)
                        + b1_ref[...], 0.0)
        g = jnp.maximum(jnp.dot(g, w2_ref[...],
                                preferred_element_type=jnp.float32)
                        + b2_ref[...], 0.0)
        o_ref[...] = jnp.dot(g, w3_ref[...],
                             preferred_element_type=jnp.float32) + b3_ref[...]

    return pl.pallas_call(
        body,
        out_shape=jax.ShapeDtypeStruct((G, 16), jnp.float32),
    )(sums, cnt, w1, b1.reshape(1, -1), w2, b2.reshape(1, -1),
      w3, b3.reshape(1, -1))


def _layer(x_in, src, dst, ea16, wq, bq, wk, bk, wv, bv, we, ws, bs, relu):
    we_h = we.reshape(4, H, C)
    z12 = jnp.zeros((HC, 12), jnp.float32)
    wq_s, bq_s, wkv_s, bkv_s = [], [], [], []
    for h in range(H):
        wqh = wq[:, h * C:(h + 1) * C]
        wp = wqh @ jnp.transpose(we_h[:, h, :])
        wq_s.append(jnp.concatenate([wqh, wp, z12], axis=1))
        bqh = bq[h * C:(h + 1) * C]
        bp = bqh @ jnp.transpose(we_h[:, h, :])
        bq_s.append(jnp.concatenate([bqh, bp, jnp.zeros((12,), jnp.float32)]))
        wkv_s.append(jnp.concatenate(
            [wk[:, h * C:(h + 1) * C], wv[:, h * C:(h + 1) * C]], axis=1))
        bkv_s.append(jnp.concatenate(
            [bk[h * C:(h + 1) * C], bv[h * C:(h + 1) * C]]))
    wq_st = jnp.stack(wq_s)
    bq_st = jnp.stack(bq_s).reshape(H, 1, QW)
    wkv_st = jnp.stack(wkv_s)
    bkv_st = jnp.stack(bkv_s).reshape(H, 1, HC)
    qp, kv, skip = _prep_call(x_in, wq_st, bq_st, wkv_st, bkv_st, ws,
                              bs.reshape(1, HC), 400)
    acc = _edge_call(qp, kv, ea16, src, dst)
    return _combine_call(acc[:N], acc[NP:NP + N],
                         skip, we_h[:, 0, :], we_h[:, 1, :], relu, 1000)


def kernel(x, edge_index, batch, edge_attr,
           l1_Wq, l1_bq, l1_Wk, l1_bk, l1_Wv, l1_bv, l1_We, l1_Ws, l1_bs,
           l2_Wq, l2_bq, l2_Wk, l2_bk, l2_Wv, l2_bv, l2_We, l2_Ws, l2_bs,
           ge_W1, ge_b1, ge_W2, ge_b2, ge_W3, ge_b3):
    src = edge_index[0]
    dst = edge_index[1]
    ea16 = jnp.concatenate(
        [edge_attr, jnp.zeros((E, 12), jnp.float32)], axis=1)
    h = _layer(x, src, dst, ea16, l1_Wq, l1_bq, l1_Wk, l1_bk, l1_Wv, l1_bv,
               l1_We, l1_Ws, l1_bs, True)
    h = _layer(h, src, dst, ea16, l2_Wq, l2_bq, l2_Wk, l2_bk, l2_Wv, l2_bv,
               l2_We, l2_Ws, l2_bs, False)
    sums, cnt = _pool_call(h, batch.reshape(N // 1000, 1, 1000), 1000)
    g = _mlp_call(sums, cnt, ge_W1, ge_b1, ge_W2, ge_b2, ge_W3, ge_b3)
    return (h, g)


# speedup vs baseline: 18.8063x; 1.2648x over previous
import functools

import jax
import jax.numpy as jnp
from jax import lax
from jax.experimental import pallas as pl
from jax.experimental.pallas import tpu as pltpu
from jax.experimental.pallas import tpu_sc as plsc

N = 10000
E = 320000
H = 2
C = 64
HC = 128
G = 64

NC = 2
NS = 16
EW = E // NS
B = 160
KC = 2
MC = 80
NBLK = EW // B
NP = 10240
RPT = NP // NS
DR = 160
QW = 80
AW = 80


def _lperm(v, perm):
    return lax.gather(
        v, perm[:, None],
        lax.GatherDimensionNumbers(offset_dims=(), collapsed_slice_dims=(0,),
                                   start_index_map=(0,)),
        (1,), mode=lax.GatherScatterMode.PROMISE_IN_BOUNDS)


def _prep_call(x, wq, bq, wkv, bkv, ws, bs, blk):
    def body(x_ref, wq_ref, bq_ref, wkv_ref, bkv_ref, ws_ref, bs_ref,
             q_ref, kv_ref, s_ref):
        xb = x_ref[...]
        q_ref[...] = jnp.dot(xb, wq_ref[0],
                             preferred_element_type=jnp.float32) + bq_ref[0]
        kv_ref[...] = jnp.dot(xb, wkv_ref[0],
                              preferred_element_type=jnp.float32) + bkv_ref[0]
        s_ref[...] = jnp.dot(xb, ws_ref[...],
                             preferred_element_type=jnp.float32) + bs_ref[...]

    nblk = N // blk
    return pl.pallas_call(
        body,
        grid=(H, nblk),
        in_specs=[
            pl.BlockSpec((blk, HC), lambda h, i: (i, 0)),
            pl.BlockSpec((1, HC, QW), lambda h, i: (h, 0, 0)),
            pl.BlockSpec((1, 1, QW), lambda h, i: (h, 0, 0)),
            pl.BlockSpec((1, HC, HC), lambda h, i: (h, 0, 0)),
            pl.BlockSpec((1, 1, HC), lambda h, i: (h, 0, 0)),
            pl.BlockSpec((HC, HC), lambda h, i: (0, 0)),
            pl.BlockSpec((1, HC), lambda h, i: (0, 0)),
        ],
        out_specs=[
            pl.BlockSpec((blk, QW), lambda h, i: (h * nblk + i, 0)),
            pl.BlockSpec((blk, HC), lambda h, i: (h * nblk + i, 0)),
            pl.BlockSpec((blk, HC), lambda h, i: (i, 0)),
        ],
        out_shape=[
            jax.ShapeDtypeStruct((H * N, QW), jnp.float32),
            jax.ShapeDtypeStruct((H * N, HC), jnp.float32),
            jax.ShapeDtypeStruct((N, HC), jnp.float32),
        ],
    )(x, wq, bq, wkv, bkv, ws, bs)


def _edge_call(qp, kv, ea16, src, dst):
    mesh = plsc.VectorSubcoreMesh(core_axis_name="c", subcore_axis_name="s")

    @functools.partial(
        pl.kernel,
        out_type=jax.ShapeDtypeStruct((NC * NP, AW), jnp.float32),
        mesh=mesh,
        scratch_types=[
            pltpu.VMEM((KC, MC), jnp.int32),
            pltpu.VMEM((KC, MC), jnp.int32),
            pltpu.VMEM((KC, MC), jnp.int32),
            pltpu.VMEM((B, QW), jnp.float32),
            pltpu.VMEM((B, HC), jnp.float32),
            pltpu.VMEM((B, 16), jnp.float32),
            pltpu.VMEM((B, AW), jnp.float32),
            pltpu.VMEM_SHARED((NP, AW), jnp.float32),
            pltpu.SemaphoreType.DMA,
        ],
        compiler_params=pltpu.CompilerParams(use_tc_tiling_on_sc=False),
    )
    def k(qp_hbm, kv_hbm, ea_hbm, src_hbm, dst_hbm, acc_hbm,
          sidx, didx, didx_raw, qrows, kvrows, earows, outrows,
          acc_sh, sem):
        core = lax.axis_index("c")
        sub = lax.axis_index("s")
        io16 = lax.iota(jnp.int32, 16)
        z16 = jnp.zeros((16,), jnp.float32)
        hoff = core * N

        def zrow(j, carry):
            for c in range(AW // 16):
                outrows[j, pl.ds(c * 16, 16)] = z16
            return carry

        lax.fori_loop(0, B, zrow, 0)
        rbase = sub * RPT
        for p in range(RPT // DR):
            pltpu.sync_copy(outrows.at[pl.ds(0, DR)],
                            acc_sh.at[pl.ds(rbase + p * DR, DR)])
        plsc.subcore_barrier()

        def blk_body(blk, carry):
            base = sub * EW + blk * B
            rowoff = base // MC
            ci = pltpu.async_copy(src_hbm.at[pl.ds(rowoff, KC)], sidx, sem)
            cd = pltpu.async_copy(dst_hbm.at[pl.ds(rowoff, KC)], didx_raw,
                                  sem)
            ce = pltpu.async_copy(ea_hbm.at[pl.ds(base, B)], earows, sem)
            ci.wait()
            cd.wait()
            for r in range(KC):
                for c in range(MC // 16):
                    sidx[r, pl.ds(c * 16, 16)] = (
                        sidx[r, pl.ds(c * 16, 16)] + hoff)
                    didx[r, pl.ds(c * 16, 16)] = (
                        didx_raw[r, pl.ds(c * 16, 16)] + hoff)
            gs = []
            for r in range(KC):
                gs.append(pltpu.async_copy(
                    qp_hbm.at[didx.at[r]],
                    qrows.at[pl.ds(r * MC, MC)], sem))
                gs.append(pltpu.async_copy(
                    kv_hbm.at[sidx.at[r]],
                    kvrows.at[pl.ds(r * MC, MC)], sem))
            ce.wait()
            for g in gs:
                g.wait()

            def edge(i, ecarry):
                a = qrows[i, pl.ds(0, 16)] * kvrows[i, pl.ds(0, 16)]
                for c in range(1, 4):
                    a = a + (qrows[i, pl.ds(c * 16, 16)]
                             * kvrows[i, pl.ds(c * 16, 16)])
                evec = earows[i, :]
                pe = qrows[i, pl.ds(C, 16)] * evec
                a = a + jnp.where(io16 < 4, pe, 0.0)
                for sh in (8, 4, 2, 1):
                    a = a + _lperm(a, io16 ^ sh)
                sv = jnp.exp(a * 0.125)
                for c in range(4):
                    outrows[i, pl.ds(c * 16, 16)] = (
                        sv * kvrows[i, pl.ds(C + c * 16, 16)])
                shead = jnp.where(io16 < 4, sv, 0.0)
                outrows[i, pl.ds(C, 16)] = (evec * shead
                                            + jnp.where(io16 == 8, sv, 0.0))
                return ecarry

            lax.fori_loop(0, B, edge, 0)
            ss = [pltpu.async_copy(outrows.at[pl.ds(r * MC, MC)],
                                   acc_sh.at[didx_raw.at[r]], sem, add=True)
                  for r in range(KC)]
            for s in ss:
                s.wait()
            return carry

        lax.fori_loop(0, NBLK, blk_body, 0)
        plsc.subcore_barrier()

        obase = core * NP + rbase
        for p in range(RPT // DR):
            pltpu.sync_copy(acc_sh.at[pl.ds(rbase + p * DR, DR)],
                            outrows.at[pl.ds(0, DR)])
            pltpu.sync_copy(outrows.at[pl.ds(0, DR)],
                            acc_hbm.at[pl.ds(obase + p * DR, DR)])

    return k(qp, kv, ea16, src.reshape(E // MC, MC),
             dst.reshape(E // MC, MC))


def _combine_call(acc0, acc1, skip, wt0, wt1, relu, blk):
    def body(a0_ref, a1_ref, s_ref, wt0_ref, wt1_ref, o_ref):
        a0 = a0_ref[...]
        a1 = a1_ref[...]
        n0 = a0[:, 0:C] + jnp.dot(a0[:, C:C + 4], wt0_ref[...],
                                  preferred_element_type=jnp.float32)
        n1 = a1[:, 0:C] + jnp.dot(a1[:, C:C + 4], wt1_ref[...],
                                  preferred_element_type=jnp.float32)
        s0 = a0[:, C + 8:C + 9]
        s1 = a1[:, C + 8:C + 9]
        h0 = jnp.where(s0 > 0.0, n0 / s0, 0.0)
        h1 = jnp.where(s1 > 0.0, n1 / s1, 0.0)
        h = jnp.concatenate([h0, h1], axis=1) + s_ref[...]
        if relu:
            h = jnp.maximum(h, 0.0)
        o_ref[...] = h

    return pl.pallas_call(
        body,
        grid=(N // blk,),
        in_specs=[
            pl.BlockSpec((blk, AW), lambda i: (i, 0)),
            pl.BlockSpec((blk, AW), lambda i: (i, 0)),
            pl.BlockSpec((blk, HC), lambda i: (i, 0)),
            pl.BlockSpec((4, C), lambda i: (0, 0)),
            pl.BlockSpec((4, C), lambda i: (0, 0)),
        ],
        out_specs=pl.BlockSpec((blk, HC), lambda i: (i, 0)),
        out_shape=jax.ShapeDtypeStruct((N, HC), jnp.float32),
    )(acc0, acc1, skip, wt0, wt1)


def _pool_call(h, batch3d, blk):
    def body(b_ref, h_ref, sum_ref, cnt_ref):
        @pl.when(pl.program_id(0) == 0)
        def _init():
            sum_ref[...] = jnp.zeros((G, HC), jnp.float32)
            cnt_ref[...] = jnp.zeros((G, HC), jnp.float32)

        oh = (b_ref[0, 0, :][None, :]
              == lax.broadcasted_iota(jnp.int32, (G, blk), 0)
              ).astype(jnp.float32)
        sum_ref[...] += jnp.dot(oh, h_ref[...],
                                preferred_element_type=jnp.float32)
        cnt_ref[...] += jnp.broadcast_to(
            jnp.sum(oh, axis=1, keepdims=True), (G, HC))

    return pl.pallas_call(
        body,
        grid=(N // blk,),
        in_specs=[
            pl.BlockSpec((1, 1, blk), lambda i: (i, 0, 0)),
            pl.BlockSpec((blk, HC), lambda i: (i, 0)),
        ],
        out_specs=[
            pl.BlockSpec((G, HC), lambda i: (0, 0)),
            pl.BlockSpec((G, HC), lambda i: (0, 0)),
        ],
        out_shape=[
            jax.ShapeDtypeStruct((G, HC), jnp.float32),
            jax.ShapeDtypeStruct((G, HC), jnp.float32),
        ],
    )(batch3d, h)


def _mlp_call(sums, cnt, w1, b1, w2, b2, w3, b3):
    def body(s_ref, c_ref, w1_ref, b1_ref, w2_ref, b2_ref, w3_ref, b3_ref,
             o_ref):
        gm = s_ref[...] / jnp.maximum(c_ref[...], 1.0)
        g = jnp.maximum(jnp.dot(gm, w1_ref[...],
                                preferred_element_type=jnp.float32)
                        + b1_ref[...], 0.0)
        g = jnp.maximum(jnp.dot(g, w2_ref[...],
                                preferred_element_type=jnp.float32)
                        + b2_ref[...], 0.0)
        o_ref[...] = jnp.dot(g, w3_ref[...],
                             preferred_element_type=jnp.float32) + b3_ref[...]

    return pl.pallas_call(
        body,
        out_shape=jax.ShapeDtypeStruct((G, 16), jnp.float32),
    )(sums, cnt, w1, b1.reshape(1, -1), w2, b2.reshape(1, -1),
      w3, b3.reshape(1, -1))


def _layer(x_in, src, dst, ea16, wq, bq, wk, bk, wv, bv, we, ws, bs, relu):
    we_h = we.reshape(4, H, C)
    z12 = jnp.zeros((HC, 12), jnp.float32)
    wq_s, bq_s, wkv_s, bkv_s = [], [], [], []
    for h in range(H):
        wqh = wq[:, h * C:(h + 1) * C]
        wp = wqh @ jnp.transpose(we_h[:, h, :])
        wq_s.append(jnp.concatenate([wqh, wp, z12], axis=1))
        bqh = bq[h * C:(h + 1) * C]
        bp = bqh @ jnp.transpose(we_h[:, h, :])
        bq_s.append(jnp.concatenate([bqh, bp, jnp.zeros((12,), jnp.float32)]))
        wkv_s.append(jnp.concatenate(
            [wk[:, h * C:(h + 1) * C], wv[:, h * C:(h + 1) * C]], axis=1))
        bkv_s.append(jnp.concatenate(
            [bk[h * C:(h + 1) * C], bv[h * C:(h + 1) * C]]))
    wq_st = jnp.stack(wq_s)
    bq_st = jnp.stack(bq_s).reshape(H, 1, QW)
    wkv_st = jnp.stack(wkv_s)
    bkv_st = jnp.stack(bkv_s).reshape(H, 1, HC)
    qp, kv, skip = _prep_call(x_in, wq_st, bq_st, wkv_st, bkv_st, ws,
                              bs.reshape(1, HC), 400)
    acc = _edge_call(qp, kv, ea16, src, dst)
    return _combine_call(acc[:N], acc[NP:NP + N],
                         skip, we_h[:, 0, :], we_h[:, 1, :], relu, 1000)


def kernel(x, edge_index, batch, edge_attr,
           l1_Wq, l1_bq, l1_Wk, l1_bk, l1_Wv, l1_bv, l1_We, l1_Ws, l1_bs,
           l2_Wq, l2_bq, l2_Wk, l2_bk, l2_Wv, l2_bv, l2_We, l2_Ws, l2_bs,
           ge_W1, ge_b1, ge_W2, ge_b2, ge_W3, ge_b3):
    src = edge_index[0]
    dst = edge_index[1]
    ea16 = jnp.concatenate(
        [edge_attr, jnp.zeros((E, 12), jnp.float32)], axis=1)
    h = _layer(x, src, dst, ea16, l1_Wq, l1_bq, l1_Wk, l1_bk, l1_Wv, l1_bv,
               l1_We, l1_Ws, l1_bs, True)
    h = _layer(h, src, dst, ea16, l2_Wq, l2_bq, l2_Wk, l2_bk, l2_Wv, l2_bv,
               l2_We, l2_Ws, l2_bs, False)
    sums, cnt = _pool_call(h, batch.reshape(N // 1000, 1, 1000), 1000)
    g = _mlp_call(sums, cnt, ge_W1, ge_b1, ge_W2, ge_b2, ge_W3, ge_b3)
    return (h, g)
